# Initial kernel scaffold; baseline (speedup 1.0000x reference)
import jax
import jax.numpy as jnp
from jax.experimental import pallas as pl


def kernel(lhs_x, lhs_edge_index, lhs_batch, rhs_x, rhs_edge_index, rhs_batch, emb, Wp, bp, Wc1, bc1, Wc2, bc2, Wf1, bf1, Wf2, bf2):
    raise NotImplementedError("write your pallas kernel here")


# speedup vs baseline: 14.7146x; 14.7146x over previous
import functools

import jax
import jax.numpy as jnp
from jax import lax
from jax.experimental import pallas as pl
from jax.experimental.pallas import tpu as pltpu
from jax.experimental.pallas import tpu_sc as plsc

N = 100000
E = 1600000
G = 64
EMB = 16
HID = 32

NC = 2
NS = 16
CH = 128
GK = 160
NG = 5
EPAD = NS * NG * GK * CH
ACC_ROWS = 100352
ZC = ACC_ROWS // NS // CH
RO = ACC_ROWS // NS

_f32 = jnp.float32
_mesh = plsc.VectorSubcoreMesh(
    core_axis_name="c", subcore_axis_name="s", num_cores=NC, num_subcores=NS)


def _zero_acc(acc, zsrc, zbuf, sid):
    pltpu.sync_copy(zsrc, zbuf)

    def body(i, carry):
        pltpu.sync_copy(zbuf, acc.at[pl.ds((sid * ZC + i) * CH, CH)])
        return carry
    lax.fori_loop(0, ZC, body, None)


def _edge_scatter(table, src_h, dst_h, out, srcbuf, dstbuf, rows0, rows1,
                  zdrain, sem0, sem1, sid):
    for g in range(NG):
        pltpu.sync_copy(src_h.at[sid, g], srcbuf)
        pltpu.sync_copy(dst_h.at[sid, g], dstbuf)
        pltpu.async_copy(table.at[srcbuf.at[0]], rows0, sem0)

        def body(j2, carry):
            je = 2 * j2
            jo = je + 1
            jn = jnp.minimum(je + 2, GK - 1)
            pltpu.async_copy(table.at[srcbuf.at[jo]], rows1, sem1)
            pltpu.make_async_copy(table.at[srcbuf.at[je]], rows0, sem0).wait()
            pltpu.sync_copy(rows0, out.at[dstbuf.at[je]], add=True)
            pltpu.async_copy(table.at[srcbuf.at[jn]], rows0, sem0)
            pltpu.make_async_copy(table.at[srcbuf.at[jo]], rows1, sem1).wait()
            pltpu.sync_copy(rows1, out.at[dstbuf.at[jo]], add=True)
            return carry
        lax.fori_loop(0, GK // 2, body, None)
        pltpu.make_async_copy(table.at[srcbuf.at[GK - 1]], rows0, sem0).wait()


def _ones_scatter(dst_h, acc, dstbuf, onesbuf, sid):
    for g in range(NG):
        pltpu.sync_copy(dst_h.at[sid, g], dstbuf)

        def body(j, carry):
            pltpu.sync_copy(onesbuf, acc.at[dstbuf.at[j]], add=True)
            return carry
        lax.fori_loop(0, GK, body, None)


@functools.partial(
    pl.kernel,
    out_type=(jax.ShapeDtypeStruct((ACC_ROWS, 8), _f32),
              jax.ShapeDtypeStruct((ACC_ROWS, 8), _f32)),
    mesh=_mesh,
    scratch_types=[
        pltpu.VMEM((GK, CH), jnp.int32),
        pltpu.VMEM((GK, CH), jnp.int32),
        pltpu.VMEM((CH, 8), _f32),
        pltpu.VMEM((CH, 8), _f32),
        pltpu.VMEM((CH, 8), _f32),
        pltpu.VMEM_SHARED((ACC_ROWS, 8), _f32),
        pltpu.SemaphoreType.DMA,
        pltpu.SemaphoreType.DMA,
    ],
    compiler_params=pltpu.CompilerParams(use_tc_tiling_on_sc=False),
)
def _conv_sc(ya, yb, src_h, dst_h, zeros8,
             agga, aggb,
             srcbuf, dstbuf, rows0, rows1, zbuf, acc, sem0, sem1):
    cid = lax.axis_index("c")
    sid = lax.axis_index("s")
    _zero_acc(acc, zeros8, zbuf, sid)
    plsc.subcore_barrier()

    @pl.when(cid == 0)
    def _():
        _edge_scatter(ya, src_h, dst_h, acc, srcbuf, dstbuf, rows0, rows1,
                      zeros8, sem0, sem1, sid)

    @pl.when(cid == 1)
    def _():
        _edge_scatter(yb, src_h, dst_h, acc, srcbuf, dstbuf, rows0, rows1,
                      zeros8, sem0, sem1, sid)

    plsc.subcore_barrier()

    @pl.when(cid == 0)
    def _():
        pltpu.sync_copy(acc.at[pl.ds(sid * RO, RO)],
                        agga.at[pl.ds(sid * RO, RO)])

    @pl.when(cid == 1)
    def _():
        pltpu.sync_copy(acc.at[pl.ds(sid * RO, RO)],
                        aggb.at[pl.ds(sid * RO, RO)])


@functools.partial(
    pl.kernel,
    out_type=(jax.ShapeDtypeStruct((ACC_ROWS, 8), _f32),
              jax.ShapeDtypeStruct((ACC_ROWS, 8), _f32)),
    mesh=_mesh,
    scratch_types=[
        pltpu.VMEM((GK, CH), jnp.int32),
        pltpu.VMEM((CH, 8), _f32),
        pltpu.VMEM((CH, 8), _f32),
        pltpu.VMEM_SHARED((ACC_ROWS, 8), _f32),
    ],
    compiler_params=pltpu.CompilerParams(use_tc_tiling_on_sc=False),
)
def _deg_sc(dstl_h, dstr_h, zeros8, ones8,
            degl, degr,
            dstbuf, onesbuf, zbuf8, acc):
    cid = lax.axis_index("c")
    sid = lax.axis_index("s")
    _zero_acc(acc, zeros8, zbuf8, sid)
    pltpu.sync_copy(ones8, onesbuf)
    plsc.subcore_barrier()

    @pl.when(cid == 0)
    def _():
        _ones_scatter(dstl_h, acc, dstbuf, onesbuf, sid)

    @pl.when(cid == 1)
    def _():
        _ones_scatter(dstr_h, acc, dstbuf, onesbuf, sid)

    plsc.subcore_barrier()

    @pl.when(cid == 0)
    def _():
        pltpu.sync_copy(acc.at[pl.ds(sid * RO, RO)],
                        degl.at[pl.ds(sid * RO, RO)])

    @pl.when(cid == 1)
    def _():
        pltpu.sync_copy(acc.at[pl.ds(sid * RO, RO)],
                        degr.at[pl.ds(sid * RO, RO)])


B = 2048
GRID = -(-N // B)
_HI = lax.Precision.HIGHEST


def _mm(a, w):
    a_hi = a.astype(jnp.bfloat16).astype(_f32)
    a_lo = a - a_hi
    return (jnp.dot(a_hi, w, preferred_element_type=_f32) +
            jnp.dot(a_lo, w, preferred_element_type=_f32))


def _dinv_of(deg_blk):
    return lax.rsqrt(jnp.maximum(deg_blk[:, 0:1] + 1.0, 1.0))


def _stage_ab(x_ref, deg_ref, emb_ref, wp_ref, bp_ref, wc1_ref,
              y0_ref, y1_ref, y2_ref, y3_ref):
    x = x_ref[...]
    ids = x[:, 0:1].astype(jnp.int32)
    h = x[:, 1:2] * wp_ref[0:1, :] + bp_ref[...]
    for k in range(10):
        h = h + jnp.where(ids == k, 1.0, 0.0) * emb_ref[k:k + 1, :]
    dinv = _dinv_of(deg_ref[...])
    y = _mm(h, wc1_ref[...]) * dinv
    y0_ref[...] = y[:, 0:8]
    y1_ref[...] = y[:, 8:16]
    y2_ref[...] = y[:, 16:24]
    y3_ref[...] = y[:, 24:32]


def _update1(a0_ref, a1_ref, a2_ref, a3_ref, y0_ref, y1_ref, y2_ref, y3_ref,
             deg_ref, bc1_ref, wc2_ref,
             z0_ref, z1_ref, z2_ref, z3_ref):
    agg = jnp.concatenate([a0_ref[...], a1_ref[...], a2_ref[...],
                           a3_ref[...]], axis=1)
    y1 = jnp.concatenate([y0_ref[...], y1_ref[...], y2_ref[...],
                          y3_ref[...]], axis=1)
    dinv = _dinv_of(deg_ref[...])
    h1 = jnp.maximum(dinv * (agg + y1) + bc1_ref[...], 0.0)
    y2 = _mm(h1, wc2_ref[...]) * dinv
    z0_ref[...] = y2[:, 0:8]
    z1_ref[...] = y2[:, 8:16]
    z2_ref[...] = y2[:, 16:24]
    z3_ref[...] = y2[:, 24:32]


def _update2_pool(a0_ref, a1_ref, a2_ref, a3_ref, y0_ref, y1_ref, y2_ref,
                  y3_ref, deg_ref, bc2_ref, batch_ref, pooled_ref):
    i = pl.program_id(0)
    agg = jnp.concatenate([a0_ref[...], a1_ref[...], a2_ref[...],
                           a3_ref[...]], axis=1)
    y2 = jnp.concatenate([y0_ref[...], y1_ref[...], y2_ref[...],
                          y3_ref[...]], axis=1)
    dinv = _dinv_of(deg_ref[...])
    h2 = jnp.maximum(dinv * (agg + y2) + bc2_ref[...], 0.0)
    h2e = jnp.concatenate([h2, jnp.ones((B, 1), _f32)], axis=1)
    rows = lax.broadcasted_iota(jnp.int32, (B, 1), 0) + i * B
    h2e = jnp.where(rows < N, h2e, 0.0)
    onehot = (batch_ref[...] ==
              lax.broadcasted_iota(jnp.int32, (1, G), 1)).astype(_f32)
    contrib = lax.dot_general(onehot, h2e, (((0,), (0,)), ((), ())),
                              precision=_HI, preferred_element_type=_f32)

    @pl.when(i == 0)
    def _():
        pooled_ref[...] = jnp.zeros_like(pooled_ref)

    pooled_ref[...] += contrib


def _head(pl_l_ref, pl_r_ref, wf1_ref, bf1_ref, wf2_ref, bf2_ref, out_ref):
    ml = pl_l_ref[:, :32] / jnp.maximum(pl_l_ref[:, 32:33], 1.0)
    mr = pl_r_ref[:, :32] / jnp.maximum(pl_r_ref[:, 32:33], 1.0)
    h = jnp.concatenate([ml, mr], axis=1)
    z = jnp.maximum(_mm(h, wf1_ref[...]) + bf1_ref[...], 0.0)
    out_ref[...] = _mm(z, wf2_ref[...]) + bf2_ref[...]


def _full(shape):
    return pl.BlockSpec(shape, lambda i: tuple(0 for _ in shape))


def _blk(shape):
    return pl.BlockSpec(shape, lambda i: (i,) + tuple(0 for _ in shape[1:]))


_Y4_OUT = dict(
    out_specs=[pl.BlockSpec((B, 8), lambda i: (i, 0)) for _ in range(4)],
    out_shape=[jax.ShapeDtypeStruct((N, 8), _f32) for _ in range(4)],
)


def _run_stage_ab(x, deg1, emb, wp, bp2, wc1):
    return pl.pallas_call(
        _stage_ab,
        grid=(GRID,),
        in_specs=[_blk((B, 2)), _blk((B, 8)), _full((10, 16)),
                  _full((1, 16)), _full((1, 16)), _full((16, 32))],
        **_Y4_OUT,
    )(x, deg1, emb, wp, bp2, wc1)


def _run_update1(aggs, ys, deg1, bc1_2, wc2):
    return pl.pallas_call(
        _update1,
        grid=(GRID,),
        in_specs=[_blk((B, 8))] * 9 +
                 [_full((1, 32)), _full((32, 32))],
        **_Y4_OUT,
    )(*aggs, *ys, deg1, bc1_2, wc2)


def _run_update2_pool(aggs, ys, deg1, bc2_2, batch2):
    return pl.pallas_call(
        _update2_pool,
        grid=(GRID,),
        in_specs=[_blk((B, 8))] * 9 +
                 [_full((1, 32)), _blk((B, 1))],
        out_specs=_full((G, HID + 1)),
        out_shape=jax.ShapeDtypeStruct((G, HID + 1), _f32),
    )(*aggs, *ys, deg1, bc2_2, batch2)


def _run_head(pooled_l, pooled_r, wf1, bf1_2, wf2, bf2_2):
    return pl.pallas_call(
        _head,
        out_shape=jax.ShapeDtypeStruct((G, 1), _f32),
    )(pooled_l, pooled_r, wf1, bf1_2, wf2, bf2_2)


def _prep_idx(edge_index):
    src = edge_index[0].astype(jnp.int32)
    dst = edge_index[1].astype(jnp.int32)
    pad = EPAD - E
    src_p = jnp.concatenate([src, jnp.zeros((pad,), jnp.int32)])
    dst_p = jnp.concatenate([dst, jnp.full((pad,), N, jnp.int32)])
    return (src_p.reshape(NS, NG, GK, CH), dst_p.reshape(NS, NG, GK, CH))


def kernel(lhs_x, lhs_edge_index, lhs_batch, rhs_x, rhs_edge_index, rhs_batch,
           emb, Wp, bp, Wc1, bc1, Wc2, bc2, Wf1, bf1, Wf2, bf2):
    srcl, dstl = _prep_idx(lhs_edge_index)
    srcr, dstr = _prep_idx(rhs_edge_index)
    zeros8 = jnp.zeros((CH, 8), _f32)
    ones8 = jnp.ones((CH, 8), _f32)
    bp2 = bp.reshape(1, 16)
    bc1_2 = bc1.reshape(1, 32)
    bc2_2 = bc2.reshape(1, 32)
    bf1_2 = bf1.reshape(1, 32)
    bf2_2 = bf2.reshape(1, 1)
    batchl2 = lhs_batch.astype(jnp.int32).reshape(N, 1)
    batchr2 = rhs_batch.astype(jnp.int32).reshape(N, 1)

    degl1, degr1 = _deg_sc(dstl, dstr, zeros8, ones8)

    def conv(ys, src, dst):
        a0, a1 = _conv_sc(ys[0], ys[1], src, dst, zeros8)
        a2, a3 = _conv_sc(ys[2], ys[3], src, dst, zeros8)
        return (a0, a1, a2, a3)

    pooled = []
    for x, src, dst, deg1, batch2 in (
            (lhs_x, srcl, dstl, degl1, batchl2),
            (rhs_x, srcr, dstr, degr1, batchr2)):
        ys1 = _run_stage_ab(x, deg1, emb, Wp, bp2, Wc1)
        aggs1 = conv(ys1, src, dst)
        ys2 = _run_update1(aggs1, ys1, deg1, bc1_2, Wc2)
        aggs2 = conv(ys2, src, dst)
        pooled.append(_run_update2_pool(aggs2, ys2, deg1, bc2_2, batch2))

    return _run_head(pooled[0], pooled[1], Wf1, bf1_2, Wf2, bf2_2)


# speedup vs baseline: 16.6673x; 1.1327x over previous
import functools

import jax
import jax.numpy as jnp
from jax import lax
from jax.experimental import pallas as pl
from jax.experimental.pallas import tpu as pltpu
from jax.experimental.pallas import tpu_sc as plsc

N = 100000
E = 1600000
G = 64
EMB = 16
HID = 32

NC = 2
NS = 16
CH = 128
GK = 160
NG = 5
EPAD = NS * NG * GK * CH
ACC_ROWS = 100352
ZC = ACC_ROWS // NS // CH
RO = ACC_ROWS // NS

_f32 = jnp.float32
_mesh = plsc.VectorSubcoreMesh(
    core_axis_name="c", subcore_axis_name="s", num_cores=NC, num_subcores=NS)


def _zero_acc(acc, zsrc, zbuf, sid):
    pltpu.sync_copy(zsrc, zbuf)

    def body(i, carry):
        pltpu.sync_copy(zbuf, acc.at[pl.ds((sid * ZC + i) * CH, CH)])
        return carry
    lax.fori_loop(0, ZC, body, None)


NBUF = 4


def _edge_scatter(table, src_h, dst_h, out, srcbuf, dstbuf, rows, sems, sid):
    for g in range(NG):
        pltpu.sync_copy(src_h.at[sid, g], srcbuf)
        pltpu.sync_copy(dst_h.at[sid, g], dstbuf)
        for b in range(NBUF):
            pltpu.async_copy(table.at[srcbuf.at[b]], rows[b], sems[b])

        def body(jq, carry):
            base = NBUF * jq
            for b in range(NBUF):
                j = base + b
                jn = jnp.minimum(j + NBUF, GK - 1)
                pltpu.make_async_copy(table.at[srcbuf.at[j]], rows[b],
                                      sems[b]).wait()
                pltpu.sync_copy(rows[b], out.at[dstbuf.at[j]], add=True)

                @pl.when(j + NBUF <= GK - 1)
                def _():
                    pltpu.async_copy(table.at[srcbuf.at[jn]], rows[b],
                                     sems[b])
            return carry
        lax.fori_loop(0, GK // NBUF, body, None)


def _ones_scatter(dst_h, acc, dstbuf, onesbuf, sid):
    for g in range(NG):
        pltpu.sync_copy(dst_h.at[sid, g], dstbuf)

        def body(j, carry):
            pltpu.sync_copy(onesbuf, acc.at[dstbuf.at[j]], add=True)
            return carry
        lax.fori_loop(0, GK, body, None)


@functools.partial(
    pl.kernel,
    out_type=(jax.ShapeDtypeStruct((ACC_ROWS, 8), _f32),
              jax.ShapeDtypeStruct((ACC_ROWS, 8), _f32)),
    mesh=_mesh,
    scratch_types=[
        pltpu.VMEM((GK, CH), jnp.int32),
        pltpu.VMEM((GK, CH), jnp.int32),
        pltpu.VMEM((CH, 8), _f32),
        pltpu.VMEM((CH, 8), _f32),
        pltpu.VMEM((CH, 8), _f32),
        pltpu.VMEM((CH, 8), _f32),
        pltpu.VMEM((CH, 8), _f32),
        pltpu.VMEM_SHARED((ACC_ROWS, 8), _f32),
        pltpu.SemaphoreType.DMA,
        pltpu.SemaphoreType.DMA,
        pltpu.SemaphoreType.DMA,
        pltpu.SemaphoreType.DMA,
    ],
    compiler_params=pltpu.CompilerParams(use_tc_tiling_on_sc=False),
)
def _conv_sc(ya, yb, src_h, dst_h, zeros8,
             agga, aggb,
             srcbuf, dstbuf, rows0, rows1, rows2, rows3, zbuf, acc,
             sem0, sem1, sem2, sem3):
    cid = lax.axis_index("c")
    sid = lax.axis_index("s")
    rows = (rows0, rows1, rows2, rows3)
    sems = (sem0, sem1, sem2, sem3)
    _zero_acc(acc, zeros8, zbuf, sid)
    plsc.subcore_barrier()

    @pl.when(cid == 0)
    def _():
        _edge_scatter(ya, src_h, dst_h, acc, srcbuf, dstbuf, rows, sems, sid)

    @pl.when(cid == 1)
    def _():
        _edge_scatter(yb, src_h, dst_h, acc, srcbuf, dstbuf, rows, sems, sid)

    plsc.subcore_barrier()

    @pl.when(cid == 0)
    def _():
        pltpu.sync_copy(acc.at[pl.ds(sid * RO, RO)],
                        agga.at[pl.ds(sid * RO, RO)])

    @pl.when(cid == 1)
    def _():
        pltpu.sync_copy(acc.at[pl.ds(sid * RO, RO)],
                        aggb.at[pl.ds(sid * RO, RO)])


@functools.partial(
    pl.kernel,
    out_type=(jax.ShapeDtypeStruct((ACC_ROWS, 8), _f32),
              jax.ShapeDtypeStruct((ACC_ROWS, 8), _f32)),
    mesh=_mesh,
    scratch_types=[
        pltpu.VMEM((GK, CH), jnp.int32),
        pltpu.VMEM((CH, 8), _f32),
        pltpu.VMEM((CH, 8), _f32),
        pltpu.VMEM_SHARED((ACC_ROWS, 8), _f32),
    ],
    compiler_params=pltpu.CompilerParams(use_tc_tiling_on_sc=False),
)
def _deg_sc(dstl_h, dstr_h, zeros8, ones8,
            degl, degr,
            dstbuf, onesbuf, zbuf8, acc):
    cid = lax.axis_index("c")
    sid = lax.axis_index("s")
    _zero_acc(acc, zeros8, zbuf8, sid)
    pltpu.sync_copy(ones8, onesbuf)
    plsc.subcore_barrier()

    @pl.when(cid == 0)
    def _():
        _ones_scatter(dstl_h, acc, dstbuf, onesbuf, sid)

    @pl.when(cid == 1)
    def _():
        _ones_scatter(dstr_h, acc, dstbuf, onesbuf, sid)

    plsc.subcore_barrier()

    @pl.when(cid == 0)
    def _():
        pltpu.sync_copy(acc.at[pl.ds(sid * RO, RO)],
                        degl.at[pl.ds(sid * RO, RO)])

    @pl.when(cid == 1)
    def _():
        pltpu.sync_copy(acc.at[pl.ds(sid * RO, RO)],
                        degr.at[pl.ds(sid * RO, RO)])


B = 2048
GRID = -(-N // B)
_HI = lax.Precision.HIGHEST


def _mm(a, w):
    a_hi = a.astype(jnp.bfloat16).astype(_f32)
    a_lo = a - a_hi
    return (jnp.dot(a_hi, w, preferred_element_type=_f32) +
            jnp.dot(a_lo, w, preferred_element_type=_f32))


def _dinv_of(deg_blk):
    return lax.rsqrt(jnp.maximum(deg_blk[:, 0:1] + 1.0, 1.0))


def _stage_ab(x_ref, deg_ref, emb_ref, wp_ref, bp_ref, wc1_ref,
              y0_ref, y1_ref, y2_ref, y3_ref):
    x = x_ref[...]
    ids = x[:, 0:1].astype(jnp.int32)
    h = x[:, 1:2] * wp_ref[0:1, :] + bp_ref[...]
    for k in range(10):
        h = h + jnp.where(ids == k, 1.0, 0.0) * emb_ref[k:k + 1, :]
    dinv = _dinv_of(deg_ref[...])
    y = _mm(h, wc1_ref[...]) * dinv
    y0_ref[...] = y[:, 0:8]
    y1_ref[...] = y[:, 8:16]
    y2_ref[...] = y[:, 16:24]
    y3_ref[...] = y[:, 24:32]


def _update1(a0_ref, a1_ref, a2_ref, a3_ref, y0_ref, y1_ref, y2_ref, y3_ref,
             deg_ref, bc1_ref, wc2_ref,
             z0_ref, z1_ref, z2_ref, z3_ref):
    agg = jnp.concatenate([a0_ref[...], a1_ref[...], a2_ref[...],
                           a3_ref[...]], axis=1)
    y1 = jnp.concatenate([y0_ref[...], y1_ref[...], y2_ref[...],
                          y3_ref[...]], axis=1)
    dinv = _dinv_of(deg_ref[...])
    h1 = jnp.maximum(dinv * (agg + y1) + bc1_ref[...], 0.0)
    y2 = _mm(h1, wc2_ref[...]) * dinv
    z0_ref[...] = y2[:, 0:8]
    z1_ref[...] = y2[:, 8:16]
    z2_ref[...] = y2[:, 16:24]
    z3_ref[...] = y2[:, 24:32]


def _update2_pool(a0_ref, a1_ref, a2_ref, a3_ref, y0_ref, y1_ref, y2_ref,
                  y3_ref, deg_ref, bc2_ref, batch_ref, pooled_ref):
    i = pl.program_id(0)
    agg = jnp.concatenate([a0_ref[...], a1_ref[...], a2_ref[...],
                           a3_ref[...]], axis=1)
    y2 = jnp.concatenate([y0_ref[...], y1_ref[...], y2_ref[...],
                          y3_ref[...]], axis=1)
    dinv = _dinv_of(deg_ref[...])
    h2 = jnp.maximum(dinv * (agg + y2) + bc2_ref[...], 0.0)
    h2e = jnp.concatenate([h2, jnp.ones((B, 1), _f32)], axis=1)
    rows = lax.broadcasted_iota(jnp.int32, (B, 1), 0) + i * B
    h2e = jnp.where(rows < N, h2e, 0.0)
    onehot = (batch_ref[...] ==
              lax.broadcasted_iota(jnp.int32, (1, G), 1)).astype(_f32)
    contrib = lax.dot_general(onehot, h2e, (((0,), (0,)), ((), ())),
                              precision=_HI, preferred_element_type=_f32)

    @pl.when(i == 0)
    def _():
        pooled_ref[...] = jnp.zeros_like(pooled_ref)

    pooled_ref[...] += contrib


def _head(pl_l_ref, pl_r_ref, wf1_ref, bf1_ref, wf2_ref, bf2_ref, out_ref):
    ml = pl_l_ref[:, :32] / jnp.maximum(pl_l_ref[:, 32:33], 1.0)
    mr = pl_r_ref[:, :32] / jnp.maximum(pl_r_ref[:, 32:33], 1.0)
    h = jnp.concatenate([ml, mr], axis=1)
    z = jnp.maximum(_mm(h, wf1_ref[...]) + bf1_ref[...], 0.0)
    out_ref[...] = _mm(z, wf2_ref[...]) + bf2_ref[...]


def _full(shape):
    return pl.BlockSpec(shape, lambda i: tuple(0 for _ in shape))


def _blk(shape):
    return pl.BlockSpec(shape, lambda i: (i,) + tuple(0 for _ in shape[1:]))


_Y4_OUT = dict(
    out_specs=[pl.BlockSpec((B, 8), lambda i: (i, 0)) for _ in range(4)],
    out_shape=[jax.ShapeDtypeStruct((N, 8), _f32) for _ in range(4)],
)


def _run_stage_ab(x, deg1, emb, wp, bp2, wc1):
    return pl.pallas_call(
        _stage_ab,
        grid=(GRID,),
        in_specs=[_blk((B, 2)), _blk((B, 8)), _full((10, 16)),
                  _full((1, 16)), _full((1, 16)), _full((16, 32))],
        **_Y4_OUT,
    )(x, deg1, emb, wp, bp2, wc1)


def _run_update1(aggs, ys, deg1, bc1_2, wc2):
    return pl.pallas_call(
        _update1,
        grid=(GRID,),
        in_specs=[_blk((B, 8))] * 9 +
                 [_full((1, 32)), _full((32, 32))],
        **_Y4_OUT,
    )(*aggs, *ys, deg1, bc1_2, wc2)


def _run_update2_pool(aggs, ys, deg1, bc2_2, batch2):
    return pl.pallas_call(
        _update2_pool,
        grid=(GRID,),
        in_specs=[_blk((B, 8))] * 9 +
                 [_full((1, 32)), _blk((B, 1))],
        out_specs=_full((G, HID + 1)),
        out_shape=jax.ShapeDtypeStruct((G, HID + 1), _f32),
    )(*aggs, *ys, deg1, bc2_2, batch2)


def _run_head(pooled_l, pooled_r, wf1, bf1_2, wf2, bf2_2):
    return pl.pallas_call(
        _head,
        out_shape=jax.ShapeDtypeStruct((G, 1), _f32),
    )(pooled_l, pooled_r, wf1, bf1_2, wf2, bf2_2)


def _prep_idx(edge_index):
    src = edge_index[0].astype(jnp.int32)
    dst = edge_index[1].astype(jnp.int32)
    pad = EPAD - E
    src_p = jnp.concatenate([src, jnp.zeros((pad,), jnp.int32)])
    dst_p = jnp.concatenate([dst, jnp.full((pad,), N, jnp.int32)])
    return (src_p.reshape(NS, NG, GK, CH), dst_p.reshape(NS, NG, GK, CH))


def kernel(lhs_x, lhs_edge_index, lhs_batch, rhs_x, rhs_edge_index, rhs_batch,
           emb, Wp, bp, Wc1, bc1, Wc2, bc2, Wf1, bf1, Wf2, bf2):
    srcl, dstl = _prep_idx(lhs_edge_index)
    srcr, dstr = _prep_idx(rhs_edge_index)
    zeros8 = jnp.zeros((CH, 8), _f32)
    ones8 = jnp.ones((CH, 8), _f32)
    bp2 = bp.reshape(1, 16)
    bc1_2 = bc1.reshape(1, 32)
    bc2_2 = bc2.reshape(1, 32)
    bf1_2 = bf1.reshape(1, 32)
    bf2_2 = bf2.reshape(1, 1)
    batchl2 = lhs_batch.astype(jnp.int32).reshape(N, 1)
    batchr2 = rhs_batch.astype(jnp.int32).reshape(N, 1)

    degl1, degr1 = _deg_sc(dstl, dstr, zeros8, ones8)

    def conv(ys, src, dst):
        a0, a1 = _conv_sc(ys[0], ys[1], src, dst, zeros8)
        a2, a3 = _conv_sc(ys[2], ys[3], src, dst, zeros8)
        return (a0, a1, a2, a3)

    pooled = []
    for x, src, dst, deg1, batch2 in (
            (lhs_x, srcl, dstl, degl1, batchl2),
            (rhs_x, srcr, dstr, degr1, batchr2)):
        ys1 = _run_stage_ab(x, deg1, emb, Wp, bp2, Wc1)
        aggs1 = conv(ys1, src, dst)
        ys2 = _run_update1(aggs1, ys1, deg1, bc1_2, Wc2)
        aggs2 = conv(ys2, src, dst)
        pooled.append(_run_update2_pool(aggs2, ys2, deg1, bc2_2, batch2))

    return _run_head(pooled[0], pooled[1], Wf1, bf1_2, Wf2, bf2_2)


# speedup vs baseline: 17.4658x; 1.0479x over previous
import functools

import jax
import jax.numpy as jnp
from jax import lax
from jax.experimental import pallas as pl
from jax.experimental.pallas import tpu as pltpu
from jax.experimental.pallas import tpu_sc as plsc

N = 100000
E = 1600000
G = 64
EMB = 16
HID = 32

NC = 2
NS = 16
CH = 128
GK = 160
NG = 5
EPAD = NS * NG * GK * CH
ACC_ROWS = 100352
ZC = ACC_ROWS // NS // CH
RO = ACC_ROWS // NS

_f32 = jnp.float32
_mesh = plsc.VectorSubcoreMesh(
    core_axis_name="c", subcore_axis_name="s", num_cores=NC, num_subcores=NS)


def _zero_acc(acc, zsrc, zbuf, sid):
    pltpu.sync_copy(zsrc, zbuf)

    def body(i, carry):
        pltpu.sync_copy(zbuf, acc.at[pl.ds((sid * ZC + i) * CH, CH)])
        return carry
    lax.fori_loop(0, ZC, body, None)


NBUF = 8


def _edge_scatter(table, src_h, dst_h, out, srcbuf, dstbuf, rows, sems, sid):
    for g in range(NG):
        pltpu.sync_copy(src_h.at[sid, g], srcbuf)
        pltpu.sync_copy(dst_h.at[sid, g], dstbuf)
        for b in range(NBUF):
            pltpu.async_copy(table.at[srcbuf.at[b]], rows[b], sems[b])

        def body(jq, carry):
            base = NBUF * jq
            for b in range(NBUF):
                j = base + b
                jn = jnp.minimum(j + NBUF, GK - 1)
                pltpu.make_async_copy(table.at[srcbuf.at[j]], rows[b],
                                      sems[b]).wait()
                pltpu.sync_copy(rows[b], out.at[dstbuf.at[j]], add=True)

                @pl.when(j + NBUF <= GK - 1)
                def _():
                    pltpu.async_copy(table.at[srcbuf.at[jn]], rows[b],
                                     sems[b])
            return carry
        lax.fori_loop(0, GK // NBUF, body, None)


def _ones_scatter(dst_h, acc, dstbuf, onesbuf, sid):
    for g in range(NG):
        pltpu.sync_copy(dst_h.at[sid, g], dstbuf)

        def body(j, carry):
            pltpu.sync_copy(onesbuf, acc.at[dstbuf.at[j]], add=True)
            return carry
        lax.fori_loop(0, GK, body, None)


@functools.partial(
    pl.kernel,
    out_type=(jax.ShapeDtypeStruct((ACC_ROWS, 8), _f32),
              jax.ShapeDtypeStruct((ACC_ROWS, 8), _f32)),
    mesh=_mesh,
    scratch_types=[
        pltpu.VMEM((GK, CH), jnp.int32),
        pltpu.VMEM((GK, CH), jnp.int32),
        *[pltpu.VMEM((CH, 8), _f32) for _ in range(NBUF)],
        pltpu.VMEM((CH, 8), _f32),
        pltpu.VMEM_SHARED((ACC_ROWS, 8), _f32),
        *[pltpu.SemaphoreType.DMA for _ in range(NBUF)],
    ],
    compiler_params=pltpu.CompilerParams(use_tc_tiling_on_sc=False),
)
def _conv_sc(ya, yb, src_h, dst_h, zeros8,
             agga, aggb,
             srcbuf, dstbuf, *rest):
    rows = rest[:NBUF]
    zbuf = rest[NBUF]
    acc = rest[NBUF + 1]
    sems = rest[NBUF + 2:]
    cid = lax.axis_index("c")
    sid = lax.axis_index("s")
    _zero_acc(acc, zeros8, zbuf, sid)
    plsc.subcore_barrier()

    @pl.when(cid == 0)
    def _():
        _edge_scatter(ya, src_h, dst_h, acc, srcbuf, dstbuf, rows, sems, sid)

    @pl.when(cid == 1)
    def _():
        _edge_scatter(yb, src_h, dst_h, acc, srcbuf, dstbuf, rows, sems, sid)

    plsc.subcore_barrier()

    @pl.when(cid == 0)
    def _():
        pltpu.sync_copy(acc.at[pl.ds(sid * RO, RO)],
                        agga.at[pl.ds(sid * RO, RO)])

    @pl.when(cid == 1)
    def _():
        pltpu.sync_copy(acc.at[pl.ds(sid * RO, RO)],
                        aggb.at[pl.ds(sid * RO, RO)])


@functools.partial(
    pl.kernel,
    out_type=(jax.ShapeDtypeStruct((ACC_ROWS, 8), _f32),
              jax.ShapeDtypeStruct((ACC_ROWS, 8), _f32)),
    mesh=_mesh,
    scratch_types=[
        pltpu.VMEM((GK, CH), jnp.int32),
        pltpu.VMEM((CH, 8), _f32),
        pltpu.VMEM((CH, 8), _f32),
        pltpu.VMEM_SHARED((ACC_ROWS, 8), _f32),
    ],
    compiler_params=pltpu.CompilerParams(use_tc_tiling_on_sc=False),
)
def _deg_sc(dstl_h, dstr_h, zeros8, ones8,
            degl, degr,
            dstbuf, onesbuf, zbuf8, acc):
    cid = lax.axis_index("c")
    sid = lax.axis_index("s")
    _zero_acc(acc, zeros8, zbuf8, sid)
    pltpu.sync_copy(ones8, onesbuf)
    plsc.subcore_barrier()

    @pl.when(cid == 0)
    def _():
        _ones_scatter(dstl_h, acc, dstbuf, onesbuf, sid)

    @pl.when(cid == 1)
    def _():
        _ones_scatter(dstr_h, acc, dstbuf, onesbuf, sid)

    plsc.subcore_barrier()

    @pl.when(cid == 0)
    def _():
        pltpu.sync_copy(acc.at[pl.ds(sid * RO, RO)],
                        degl.at[pl.ds(sid * RO, RO)])

    @pl.when(cid == 1)
    def _():
        pltpu.sync_copy(acc.at[pl.ds(sid * RO, RO)],
                        degr.at[pl.ds(sid * RO, RO)])


B = 2048
GRID = -(-N // B)
_HI = lax.Precision.HIGHEST


def _mm(a, w):
    a_hi = a.astype(jnp.bfloat16).astype(_f32)
    a_lo = a - a_hi
    return (jnp.dot(a_hi, w, preferred_element_type=_f32) +
            jnp.dot(a_lo, w, preferred_element_type=_f32))


def _dinv_of(deg_blk):
    return lax.rsqrt(jnp.maximum(deg_blk[:, 0:1] + 1.0, 1.0))


def _stage_ab(x_ref, deg_ref, emb_ref, wp_ref, bp_ref, wc1_ref,
              y0_ref, y1_ref, y2_ref, y3_ref):
    x = x_ref[...]
    ids = x[:, 0:1].astype(jnp.int32)
    h = x[:, 1:2] * wp_ref[0:1, :] + bp_ref[...]
    for k in range(10):
        h = h + jnp.where(ids == k, 1.0, 0.0) * emb_ref[k:k + 1, :]
    dinv = _dinv_of(deg_ref[...])
    y = _mm(h, wc1_ref[...]) * dinv
    y0_ref[...] = y[:, 0:8]
    y1_ref[...] = y[:, 8:16]
    y2_ref[...] = y[:, 16:24]
    y3_ref[...] = y[:, 24:32]


def _update1(a0_ref, a1_ref, a2_ref, a3_ref, y0_ref, y1_ref, y2_ref, y3_ref,
             deg_ref, bc1_ref, wc2_ref,
             z0_ref, z1_ref, z2_ref, z3_ref):
    agg = jnp.concatenate([a0_ref[...], a1_ref[...], a2_ref[...],
                           a3_ref[...]], axis=1)
    y1 = jnp.concatenate([y0_ref[...], y1_ref[...], y2_ref[...],
                          y3_ref[...]], axis=1)
    dinv = _dinv_of(deg_ref[...])
    h1 = jnp.maximum(dinv * (agg + y1) + bc1_ref[...], 0.0)
    y2 = _mm(h1, wc2_ref[...]) * dinv
    z0_ref[...] = y2[:, 0:8]
    z1_ref[...] = y2[:, 8:16]
    z2_ref[...] = y2[:, 16:24]
    z3_ref[...] = y2[:, 24:32]


def _update2_pool(a0_ref, a1_ref, a2_ref, a3_ref, y0_ref, y1_ref, y2_ref,
                  y3_ref, deg_ref, bc2_ref, batch_ref, pooled_ref):
    i = pl.program_id(0)
    agg = jnp.concatenate([a0_ref[...], a1_ref[...], a2_ref[...],
                           a3_ref[...]], axis=1)
    y2 = jnp.concatenate([y0_ref[...], y1_ref[...], y2_ref[...],
                          y3_ref[...]], axis=1)
    dinv = _dinv_of(deg_ref[...])
    h2 = jnp.maximum(dinv * (agg + y2) + bc2_ref[...], 0.0)
    h2e = jnp.concatenate([h2, jnp.ones((B, 1), _f32)], axis=1)
    rows = lax.broadcasted_iota(jnp.int32, (B, 1), 0) + i * B
    h2e = jnp.where(rows < N, h2e, 0.0)
    onehot = (batch_ref[...] ==
              lax.broadcasted_iota(jnp.int32, (1, G), 1)).astype(_f32)
    contrib = lax.dot_general(onehot, h2e, (((0,), (0,)), ((), ())),
                              precision=_HI, preferred_element_type=_f32)

    @pl.when(i == 0)
    def _():
        pooled_ref[...] = jnp.zeros_like(pooled_ref)

    pooled_ref[...] += contrib


def _head(pl_l_ref, pl_r_ref, wf1_ref, bf1_ref, wf2_ref, bf2_ref, out_ref):
    ml = pl_l_ref[:, :32] / jnp.maximum(pl_l_ref[:, 32:33], 1.0)
    mr = pl_r_ref[:, :32] / jnp.maximum(pl_r_ref[:, 32:33], 1.0)
    h = jnp.concatenate([ml, mr], axis=1)
    z = jnp.maximum(_mm(h, wf1_ref[...]) + bf1_ref[...], 0.0)
    out_ref[...] = _mm(z, wf2_ref[...]) + bf2_ref[...]


def _full(shape):
    return pl.BlockSpec(shape, lambda i: tuple(0 for _ in shape))


def _blk(shape):
    return pl.BlockSpec(shape, lambda i: (i,) + tuple(0 for _ in shape[1:]))


_Y4_OUT = dict(
    out_specs=[pl.BlockSpec((B, 8), lambda i: (i, 0)) for _ in range(4)],
    out_shape=[jax.ShapeDtypeStruct((N, 8), _f32) for _ in range(4)],
)


def _run_stage_ab(x, deg1, emb, wp, bp2, wc1):
    return pl.pallas_call(
        _stage_ab,
        grid=(GRID,),
        in_specs=[_blk((B, 2)), _blk((B, 8)), _full((10, 16)),
                  _full((1, 16)), _full((1, 16)), _full((16, 32))],
        **_Y4_OUT,
    )(x, deg1, emb, wp, bp2, wc1)


def _run_update1(aggs, ys, deg1, bc1_2, wc2):
    return pl.pallas_call(
        _update1,
        grid=(GRID,),
        in_specs=[_blk((B, 8))] * 9 +
                 [_full((1, 32)), _full((32, 32))],
        **_Y4_OUT,
    )(*aggs, *ys, deg1, bc1_2, wc2)


def _run_update2_pool(aggs, ys, deg1, bc2_2, batch2):
    return pl.pallas_call(
        _update2_pool,
        grid=(GRID,),
        in_specs=[_blk((B, 8))] * 9 +
                 [_full((1, 32)), _blk((B, 1))],
        out_specs=_full((G, HID + 1)),
        out_shape=jax.ShapeDtypeStruct((G, HID + 1), _f32),
    )(*aggs, *ys, deg1, bc2_2, batch2)


def _run_head(pooled_l, pooled_r, wf1, bf1_2, wf2, bf2_2):
    return pl.pallas_call(
        _head,
        out_shape=jax.ShapeDtypeStruct((G, 1), _f32),
    )(pooled_l, pooled_r, wf1, bf1_2, wf2, bf2_2)


def _prep_idx(edge_index):
    src = edge_index[0].astype(jnp.int32)
    dst = edge_index[1].astype(jnp.int32)
    pad = EPAD - E
    src_p = jnp.concatenate([src, jnp.zeros((pad,), jnp.int32)])
    dst_p = jnp.concatenate([dst, jnp.full((pad,), N, jnp.int32)])
    return (src_p.reshape(NS, NG, GK, CH), dst_p.reshape(NS, NG, GK, CH))


def kernel(lhs_x, lhs_edge_index, lhs_batch, rhs_x, rhs_edge_index, rhs_batch,
           emb, Wp, bp, Wc1, bc1, Wc2, bc2, Wf1, bf1, Wf2, bf2):
    srcl, dstl = _prep_idx(lhs_edge_index)
    srcr, dstr = _prep_idx(rhs_edge_index)
    zeros8 = jnp.zeros((CH, 8), _f32)
    ones8 = jnp.ones((CH, 8), _f32)
    bp2 = bp.reshape(1, 16)
    bc1_2 = bc1.reshape(1, 32)
    bc2_2 = bc2.reshape(1, 32)
    bf1_2 = bf1.reshape(1, 32)
    bf2_2 = bf2.reshape(1, 1)
    batchl2 = lhs_batch.astype(jnp.int32).reshape(N, 1)
    batchr2 = rhs_batch.astype(jnp.int32).reshape(N, 1)

    degl1, degr1 = _deg_sc(dstl, dstr, zeros8, ones8)

    def conv(ys, src, dst):
        a0, a1 = _conv_sc(ys[0], ys[1], src, dst, zeros8)
        a2, a3 = _conv_sc(ys[2], ys[3], src, dst, zeros8)
        return (a0, a1, a2, a3)

    pooled = []
    for x, src, dst, deg1, batch2 in (
            (lhs_x, srcl, dstl, degl1, batchl2),
            (rhs_x, srcr, dstr, degr1, batchr2)):
        ys1 = _run_stage_ab(x, deg1, emb, Wp, bp2, Wc1)
        aggs1 = conv(ys1, src, dst)
        ys2 = _run_update1(aggs1, ys1, deg1, bc1_2, Wc2)
        aggs2 = conv(ys2, src, dst)
        pooled.append(_run_update2_pool(aggs2, ys2, deg1, bc2_2, batch2))

    return _run_head(pooled[0], pooled[1], Wf1, bf1_2, Wf2, bf2_2)
